# baseline (speedup 1.0000x reference)
import functools

import jax
import jax.numpy as jnp
from jax import lax
from jax.experimental import pallas as pl
from jax.experimental.pallas import tpu as pltpu
from jax.experimental.pallas import tpu_sc as plsc

K = 10
REG = 1e-06
BLK = 128
SEG = 4
_NW = 32
_NC = 2


def _knn_block(x_ref, nbr_ref):
    i = pl.program_id(0)
    X = x_ref[...]
    N = X.shape[0]
    xb = x_ref[pl.ds(i * BLK, BLK), :]

    G = lax.dot_general(xb, X, (((1,), (1,)), ((), ())),
                        preferred_element_type=jnp.float32)
    sq_all = jnp.sum(X * X, axis=1)[None, :]
    sq_b = jnp.sum(xb * xb, axis=1)[:, None]
    D2 = jnp.maximum(sq_b + sq_all - 2.0 * G, 0.0)

    col = lax.broadcasted_iota(jnp.int32, (BLK, N), 1)
    key = (lax.bitcast_convert_type(D2, jnp.int32) & (-N)) | col
    imax = jnp.iinfo(jnp.int32).max
    nbrs = []
    mprev = jnp.full((BLK, 1), jnp.iinfo(jnp.int32).min, jnp.int32)
    for t in range(K + 1):
        m = jnp.min(jnp.where(key > mprev, key, imax),
                    axis=1, keepdims=True)
        if t > 0:
            nbrs.append(m & (N - 1))
        mprev = m

    nbr_ref[...] = jnp.transpose(jnp.concatenate(nbrs, axis=1))


def _sc_gather(idx_hbm, x_hbm, z_hbm, xn_hbm, zn_hbm,
               idx_v, xr_v, zr_v, s1, s2):
    wid = lax.axis_index("s") * _NC + lax.axis_index("c")
    chunk = idx_v.shape[0]
    base = wid * chunk
    pltpu.sync_copy(idx_hbm.at[pl.ds(base, chunk)], idx_v)
    cp1 = pltpu.async_copy(x_hbm.at[idx_v], xr_v, s1)
    cp2 = pltpu.async_copy(z_hbm.at[idx_v], zr_v, s2)
    cp1.wait()
    cp2.wait()
    pltpu.sync_copy(xr_v, xn_hbm.at[pl.ds(base, chunk)])
    pltpu.sync_copy(zr_v, zn_hbm.at[pl.ds(base, chunk)])


def _lle_solve_block(xb_ref, zb_ref, xn_ref, zn_ref, p_ref, out_ref,
                     *, finalize, denom):
    i = pl.program_id(0)
    nblk = pl.num_programs(0)
    xb = xb_ref[...]
    zb = zb_ref[...]
    dz = zb.shape[1]

    diffs = [xn_ref[a] - xb for a in range(K)]
    zn = [zn_ref[a][:, :dz] for a in range(K)]

    ent = {}
    for a in range(K):
        for b in range(a, K):
            cab = jnp.sum(diffs[a] * diffs[b], axis=1, keepdims=True)
            if a == b:
                cab = cab + REG
            ent[(a, b)] = cab
            ent[(b, a)] = cab
    rows = [jnp.transpose(
        jnp.concatenate([ent[(a, b)] for b in range(K)], axis=1))
        for a in range(K)]
    rhs = [jnp.ones((1, BLK), jnp.float32) for _ in range(K)]

    for j in range(K):
        inv = 1.0 / rows[j][j:j + 1, :]
        for r in range(K):
            if r == j:
                continue
            f = rows[r][j:j + 1, :] * inv
            rows[r] = rows[r] - f * rows[j]
            rhs[r] = rhs[r] - f * rhs[j]
    w = [rhs[a] / rows[a][a:a + 1, :] for a in range(K)]
    wsum = functools.reduce(lambda p, q: p + q, w)
    wt = jnp.transpose(
        jnp.concatenate([w[a] / wsum for a in range(K)], axis=0))
    recon = functools.reduce(
        lambda p, q: p + q, [wt[:, a:a + 1] * zn[a] for a in range(K)])

    partial = jnp.sum((recon - zb) ** 2).reshape(1, 1)
    prev = jnp.where(i == 0, p_ref[...], out_ref[...])
    acc = prev + partial

    @pl.when(i < nblk - 1)
    def _acc():
        out_ref[...] = acc

    @pl.when(i == nblk - 1)
    def _fin():
        out_ref[...] = acc / denom if finalize else acc


def kernel(X, Z):
    n, d = X.shape
    dz = Z.shape[1]
    nblk = n // BLK

    nbr_t = pl.pallas_call(
        _knn_block,
        grid=(nblk,),
        out_shape=jax.ShapeDtypeStruct((K, n), jnp.int32),
        out_specs=pl.BlockSpec((K, BLK), lambda i: (0, i)),
    )(X)

    zp = jnp.pad(Z, ((0, 0), (0, d - dz)))
    seg_pts = n // SEG
    seg_nblk = seg_pts // BLK
    chunk = (K * seg_pts) // _NW
    mesh = plsc.VectorSubcoreMesh(core_axis_name="c", subcore_axis_name="s")
    sc_gather = pl.kernel(
        _sc_gather,
        mesh=mesh,
        out_type=(jax.ShapeDtypeStruct((K * seg_pts, d), jnp.float32),
                  jax.ShapeDtypeStruct((K * seg_pts, d), jnp.float32)),
        scratch_types=[
            pltpu.VMEM((chunk,), jnp.int32),
            pltpu.VMEM((chunk, d), jnp.float32),
            pltpu.VMEM((chunk, d), jnp.float32),
            pltpu.SemaphoreType.DMA,
            pltpu.SemaphoreType.DMA,
        ],
    )

    gathered = []
    for s in range(SEG):
        lo = s * seg_pts
        idx_s = nbr_t[:, lo:lo + seg_pts].reshape(-1)
        gathered.append(sc_gather(idx_s, X, zp))

    part = jnp.zeros((1, 1), jnp.float32)
    for s in range(SEG):
        lo_blk = (s * seg_pts) // BLK
        xn_s, zn_s = gathered[s]
        part = pl.pallas_call(
            functools.partial(_lle_solve_block,
                              finalize=(s == SEG - 1),
                              denom=float(n * dz)),
            grid=(seg_nblk,),
            out_shape=jax.ShapeDtypeStruct((1, 1), jnp.float32),
            in_specs=[
                pl.BlockSpec((BLK, d), lambda i, o=lo_blk: (o + i, 0)),
                pl.BlockSpec((BLK, dz), lambda i, o=lo_blk: (o + i, 0)),
                pl.BlockSpec((K, BLK, d), lambda i: (0, i, 0)),
                pl.BlockSpec((K, BLK, d), lambda i: (0, i, 0)),
                pl.BlockSpec((1, 1), lambda i: (0, 0)),
            ],
            out_specs=pl.BlockSpec((1, 1), lambda i: (0, 0)),
        )(X, Z, xn_s.reshape(K, seg_pts, d), zn_s.reshape(K, seg_pts, d),
          part)
    return part.reshape(())


# speedup vs baseline: 1.3314x; 1.3314x over previous
import functools

import jax
import jax.numpy as jnp
from jax import lax
from jax.experimental import pallas as pl

K = 10
REG = 1e-06
BLK = 128


def _lle_block(x_ref, z_ref, out_ref):
    i = pl.program_id(0)
    nblk = pl.num_programs(0)
    X = x_ref[...]
    Z = z_ref[...]
    N = X.shape[0]
    xb = x_ref[pl.ds(i * BLK, BLK), :]
    zb = z_ref[pl.ds(i * BLK, BLK), :]

    G = lax.dot_general(xb, X, (((1,), (1,)), ((), ())),
                        preferred_element_type=jnp.float32)
    sq_all = jnp.sum(X * X, axis=1)[None, :]
    sq_b = jnp.sum(xb * xb, axis=1)[:, None]
    D2 = jnp.maximum(sq_b + sq_all - 2.0 * G, 0.0)

    col = lax.broadcasted_iota(jnp.int32, (BLK, N), 1)
    key = (lax.bitcast_convert_type(D2, jnp.int32) & (-N)) | col
    imax = jnp.iinfo(jnp.int32).max
    nbrs = []
    mprev = jnp.full((BLK, 1), jnp.iinfo(jnp.int32).min, jnp.int32)
    for t in range(K + 1):
        m = jnp.min(jnp.where(key > mprev, key, imax),
                    axis=1, keepdims=True)
        if t > 0:
            nbrs.append(m & (N - 1))
        mprev = m

    diffs = []
    zn = []
    for a in range(K):
        onehot = (col == nbrs[a]).astype(jnp.float32)
        xn_a = lax.dot_general(onehot, X, (((1,), (0,)), ((), ())),
                               preferred_element_type=jnp.float32)
        zn_a = lax.dot_general(onehot, Z, (((1,), (0,)), ((), ())),
                               preferred_element_type=jnp.float32)
        diffs.append(xn_a - xb)
        zn.append(zn_a)

    ent = {}
    for a in range(K):
        for b in range(a, K):
            cab = jnp.sum(diffs[a] * diffs[b], axis=1, keepdims=True)
            if a == b:
                cab = cab + REG
            ent[(a, b)] = cab
            ent[(b, a)] = cab
    rows = [jnp.transpose(
        jnp.concatenate([ent[(a, b)] for b in range(K)], axis=1))
        for a in range(K)]
    rhs = [jnp.ones((1, BLK), jnp.float32) for _ in range(K)]

    for j in range(K):
        inv = 1.0 / rows[j][j:j + 1, :]
        for r in range(K):
            if r == j:
                continue
            f = rows[r][j:j + 1, :] * inv
            rows[r] = rows[r] - f * rows[j]
            rhs[r] = rhs[r] - f * rhs[j]
    w = [rhs[a] / rows[a][a:a + 1, :] for a in range(K)]
    wsum = functools.reduce(lambda p, q: p + q, w)
    wt = jnp.transpose(
        jnp.concatenate([w[a] / wsum for a in range(K)], axis=0))
    recon = functools.reduce(
        lambda p, q: p + q, [wt[:, a:a + 1] * zn[a] for a in range(K)])

    partial = jnp.sum((recon - zb) ** 2).reshape(1, 1)

    @pl.when(i == 0)
    def _init():
        out_ref[...] = jnp.zeros((1, 1), jnp.float32)

    acc = out_ref[...] + partial

    @pl.when(i < nblk - 1)
    def _acc():
        out_ref[...] = acc

    @pl.when(i == nblk - 1)
    def _fin():
        out_ref[...] = acc / (N * Z.shape[1])


def kernel(X, Z):
    n = X.shape[0]
    out = pl.pallas_call(
        _lle_block,
        grid=(n // BLK,),
        out_shape=jax.ShapeDtypeStruct((1, 1), jnp.float32),
    )(X, Z)
    return out.reshape(())


# speedup vs baseline: 1.5591x; 1.1710x over previous
import functools

import jax
import jax.numpy as jnp
from jax import lax
from jax.experimental import pallas as pl

K = 10
REG = 1e-06
BLK = 256


def _lle_block(x_ref, z_ref, out_ref):
    i = pl.program_id(0)
    nblk = pl.num_programs(0)
    X = x_ref[...]
    Z = z_ref[...]
    N = X.shape[0]
    xb = x_ref[pl.ds(i * BLK, BLK), :]
    zb = z_ref[pl.ds(i * BLK, BLK), :]

    G = lax.dot_general(xb, X, (((1,), (1,)), ((), ())),
                        preferred_element_type=jnp.float32)
    sq_all = jnp.sum(X * X, axis=1)[None, :]
    sq_b = jnp.sum(xb * xb, axis=1)[:, None]
    D2 = jnp.maximum(sq_b + sq_all - 2.0 * G, 0.0)

    col = lax.broadcasted_iota(jnp.int32, (BLK, N), 1)
    key = (lax.bitcast_convert_type(D2, jnp.int32) & (-N)) | col
    imax = jnp.iinfo(jnp.int32).max
    nbrs = []
    mprev = jnp.full((BLK, 1), jnp.iinfo(jnp.int32).min, jnp.int32)
    for t in range(K + 1):
        m = jnp.min(jnp.where(key > mprev, key, imax),
                    axis=1, keepdims=True)
        if t > 0:
            nbrs.append(m & (N - 1))
        mprev = m

    diffs = []
    zn = []
    for a in range(K):
        onehot = (col == nbrs[a]).astype(jnp.float32)
        xn_a = lax.dot_general(onehot, X, (((1,), (0,)), ((), ())),
                               preferred_element_type=jnp.float32)
        zn_a = lax.dot_general(onehot, Z, (((1,), (0,)), ((), ())),
                               preferred_element_type=jnp.float32)
        diffs.append(xn_a - xb)
        zn.append(zn_a)

    ent = {}
    for a in range(K):
        for b in range(a, K):
            cab = jnp.sum(diffs[a] * diffs[b], axis=1, keepdims=True)
            if a == b:
                cab = cab + REG
            ent[(a, b)] = cab
            ent[(b, a)] = cab
    rows = [jnp.transpose(
        jnp.concatenate([ent[(a, b)] for b in range(K)], axis=1))
        for a in range(K)]
    rhs = [jnp.ones((1, BLK), jnp.float32) for _ in range(K)]

    for j in range(K):
        inv = 1.0 / rows[j][j:j + 1, :]
        for r in range(K):
            if r == j:
                continue
            f = rows[r][j:j + 1, :] * inv
            rows[r] = rows[r] - f * rows[j]
            rhs[r] = rhs[r] - f * rhs[j]
    w = [rhs[a] / rows[a][a:a + 1, :] for a in range(K)]
    wsum = functools.reduce(lambda p, q: p + q, w)
    wt = jnp.transpose(
        jnp.concatenate([w[a] / wsum for a in range(K)], axis=0))
    recon = functools.reduce(
        lambda p, q: p + q, [wt[:, a:a + 1] * zn[a] for a in range(K)])

    partial = jnp.sum((recon - zb) ** 2).reshape(1, 1)

    @pl.when(i == 0)
    def _init():
        out_ref[...] = jnp.zeros((1, 1), jnp.float32)

    acc = out_ref[...] + partial

    @pl.when(i < nblk - 1)
    def _acc():
        out_ref[...] = acc

    @pl.when(i == nblk - 1)
    def _fin():
        out_ref[...] = acc / (N * Z.shape[1])


def kernel(X, Z):
    n = X.shape[0]
    out = pl.pallas_call(
        _lle_block,
        grid=(n // BLK,),
        out_shape=jax.ShapeDtypeStruct((1, 1), jnp.float32),
    )(X, Z)
    return out.reshape(())


# speedup vs baseline: 1.5870x; 1.0179x over previous
import functools

import jax
import jax.numpy as jnp
from jax import lax
from jax.experimental import pallas as pl

K = 10
REG = 1e-06
BLK = 512


def _lle_block(x_ref, z_ref, out_ref):
    i = pl.program_id(0)
    nblk = pl.num_programs(0)
    X = x_ref[...]
    Z = z_ref[...]
    N = X.shape[0]
    xb = x_ref[pl.ds(i * BLK, BLK), :]
    zb = z_ref[pl.ds(i * BLK, BLK), :]

    G = lax.dot_general(xb, X, (((1,), (1,)), ((), ())),
                        preferred_element_type=jnp.float32)
    sq_all = jnp.sum(X * X, axis=1)[None, :]
    sq_b = jnp.sum(xb * xb, axis=1)[:, None]
    D2 = jnp.maximum(sq_b + sq_all - 2.0 * G, 0.0)

    col = lax.broadcasted_iota(jnp.int32, (BLK, N), 1)
    key = (lax.bitcast_convert_type(D2, jnp.int32) & (-N)) | col
    imax = jnp.iinfo(jnp.int32).max
    nbrs = []
    mprev = jnp.full((BLK, 1), jnp.iinfo(jnp.int32).min, jnp.int32)
    for t in range(K + 1):
        m = jnp.min(jnp.where(key > mprev, key, imax),
                    axis=1, keepdims=True)
        if t > 0:
            nbrs.append(m & (N - 1))
        mprev = m

    diffs = []
    zn = []
    for a in range(K):
        onehot = (col == nbrs[a]).astype(jnp.float32)
        xn_a = lax.dot_general(onehot, X, (((1,), (0,)), ((), ())),
                               preferred_element_type=jnp.float32)
        zn_a = lax.dot_general(onehot, Z, (((1,), (0,)), ((), ())),
                               preferred_element_type=jnp.float32)
        diffs.append(xn_a - xb)
        zn.append(zn_a)

    ent = {}
    for a in range(K):
        for b in range(a, K):
            cab = jnp.sum(diffs[a] * diffs[b], axis=1, keepdims=True)
            if a == b:
                cab = cab + REG
            ent[(a, b)] = cab
            ent[(b, a)] = cab
    rows = [jnp.transpose(
        jnp.concatenate([ent[(a, b)] for b in range(K)], axis=1))
        for a in range(K)]
    rhs = [jnp.ones((1, BLK), jnp.float32) for _ in range(K)]

    for j in range(K):
        inv = 1.0 / rows[j][j:j + 1, :]
        for r in range(K):
            if r == j:
                continue
            f = rows[r][j:j + 1, :] * inv
            rows[r] = rows[r] - f * rows[j]
            rhs[r] = rhs[r] - f * rhs[j]
    w = [rhs[a] / rows[a][a:a + 1, :] for a in range(K)]
    wsum = functools.reduce(lambda p, q: p + q, w)
    wt = jnp.transpose(
        jnp.concatenate([w[a] / wsum for a in range(K)], axis=0))
    recon = functools.reduce(
        lambda p, q: p + q, [wt[:, a:a + 1] * zn[a] for a in range(K)])

    partial = jnp.sum((recon - zb) ** 2).reshape(1, 1)

    @pl.when(i == 0)
    def _init():
        out_ref[...] = jnp.zeros((1, 1), jnp.float32)

    acc = out_ref[...] + partial

    @pl.when(i < nblk - 1)
    def _acc():
        out_ref[...] = acc

    @pl.when(i == nblk - 1)
    def _fin():
        out_ref[...] = acc / (N * Z.shape[1])


def kernel(X, Z):
    n = X.shape[0]
    out = pl.pallas_call(
        _lle_block,
        grid=(n // BLK,),
        out_shape=jax.ShapeDtypeStruct((1, 1), jnp.float32),
    )(X, Z)
    return out.reshape(())
